# Initial kernel scaffold; baseline (speedup 1.0000x reference)
import functools

import jax
import jax.numpy as jnp
from jax.experimental import pallas as pl
from jax.experimental.pallas import tpu as pltpu

IOU_T = jnp.float32(0.5)
SCORE_T = jnp.float32(0.05)
MAXDET = 100
NEG = jnp.float32(-1e30)

LANES = 128
OD = 128


def _nms_tc_body(sc_ref, x1_ref, y1_ref, x2_ref, y2_ref, cls_ref,
                 oidx_ref, osc_ref, ox1_ref, oy1_ref, ox2_ref, oy2_ref,
                 ocls_ref, onum_ref, act_ref):
    rows = sc_ref.shape[1]
    x1 = x1_ref[0]
    y1 = y1_ref[0]
    x2 = x2_ref[0]
    y2 = y2_ref[0]
    cls = cls_ref[0]
    sc = sc_ref[0]
    act_ref[:, :] = jnp.where(sc > SCORE_T, sc, NEG)
    area2 = (x2 - x1) * (y2 - y1)
    r = jax.lax.broadcasted_iota(jnp.int32, (rows, LANES), 0)
    c = jax.lax.broadcasted_iota(jnp.int32, (rows, LANES), 1)
    lin = r * LANES + c
    oiota = jax.lax.broadcasted_iota(jnp.int32, (1, OD), 1)

    def body(t, carry):
        oidx, osc, ox1, oy1, ox2, oy2, ocls, num = carry
        act = act_ref[:, :]
        maxv = jnp.max(act)
        keep = maxv > NEG * 0.5
        idx = jnp.min(jnp.where(act == maxv, lin, jnp.int32(2**30)))
        sel = lin == idx
        zf = jnp.float32(0.0)
        wx1 = jnp.sum(jnp.where(sel, x1, zf))
        wy1 = jnp.sum(jnp.where(sel, y1, zf))
        wx2 = jnp.sum(jnp.where(sel, x2, zf))
        wy2 = jnp.sum(jnp.where(sel, y2, zf))
        wcls = jnp.sum(jnp.where(sel, cls, 0))
        xx1 = jnp.maximum(wx1, x1)
        yy1 = jnp.maximum(wy1, y1)
        xx2 = jnp.minimum(wx2, x2)
        yy2 = jnp.minimum(wy2, y2)
        inter = jnp.maximum(xx2 - xx1, zf) * jnp.maximum(yy2 - yy1, zf)
        area1 = (wx2 - wx1) * (wy2 - wy1)
        iou = inter / (area1 + area2 - inter + jnp.float32(1e-9))
        supp = (iou > IOU_T) | sel
        act_ref[:, :] = jnp.where(supp, NEG, act)
        m = oiota == t
        ki = keep.astype(jnp.int32)
        oidx = jnp.where(m, jnp.where(keep, idx, -1), oidx)
        osc = jnp.where(m, jnp.where(keep, maxv, zf), osc)
        ox1 = jnp.where(m, jnp.where(keep, wx1, zf), ox1)
        oy1 = jnp.where(m, jnp.where(keep, wy1, zf), oy1)
        ox2 = jnp.where(m, jnp.where(keep, wx2, zf), ox2)
        oy2 = jnp.where(m, jnp.where(keep, wy2, zf), oy2)
        ocls = jnp.where(m, jnp.where(keep, wcls, -1), ocls)
        return (oidx, osc, ox1, oy1, ox2, oy2, ocls, num + ki)

    init = (jnp.full((1, OD), -1, jnp.int32),
            jnp.zeros((1, OD), jnp.float32),
            jnp.zeros((1, OD), jnp.float32),
            jnp.zeros((1, OD), jnp.float32),
            jnp.zeros((1, OD), jnp.float32),
            jnp.zeros((1, OD), jnp.float32),
            jnp.full((1, OD), -1, jnp.int32),
            jnp.int32(0))
    oidx, osc, ox1, oy1, ox2, oy2, ocls, num = jax.lax.fori_loop(
        0, MAXDET, body, init)
    oidx_ref[0] = oidx
    osc_ref[0] = osc
    ox1_ref[0] = ox1
    oy1_ref[0] = oy1
    ox2_ref[0] = ox2
    oy2_ref[0] = oy2
    ocls_ref[0] = ocls
    onum_ref[0] = jnp.full((1, OD), num, jnp.int32)


@jax.jit
def kernel(scores, boxes, classes):
    B, N = scores.shape
    rows = (N + LANES - 1) // LANES
    rows = ((rows + 7) // 8) * 8
    NP = rows * LANES
    pad = NP - N
    scp = jnp.pad(scores, ((0, 0), (0, pad)), constant_values=-1.0)
    clsp = jnp.pad(classes, ((0, 0), (0, pad)))
    bx = jnp.pad(boxes, ((0, 0), (0, pad), (0, 0)))
    sc = scp.reshape(B, rows, LANES)
    x1 = bx[:, :, 0].reshape(B, rows, LANES)
    y1 = bx[:, :, 1].reshape(B, rows, LANES)
    x2 = bx[:, :, 2].reshape(B, rows, LANES)
    y2 = bx[:, :, 3].reshape(B, rows, LANES)
    cl = clsp.reshape(B, rows, LANES)

    in_spec = pl.BlockSpec((1, rows, LANES), lambda b: (b, 0, 0))
    out_spec = pl.BlockSpec((1, 1, OD), lambda b: (b, 0, 0))
    of = jax.ShapeDtypeStruct((B, 1, OD), jnp.float32)
    oi = jax.ShapeDtypeStruct((B, 1, OD), jnp.int32)
    outs = pl.pallas_call(
        _nms_tc_body,
        grid=(B,),
        in_specs=[in_spec] * 6,
        out_specs=[out_spec] * 8,
        out_shape=[oi, of, of, of, of, of, oi, oi],
        scratch_shapes=[pltpu.VMEM((rows, LANES), jnp.float32)],
    )(sc, x1, y1, x2, y2, cl)
    oidx, osc, ox1, oy1, ox2, oy2, ocls, onum = outs
    idxs = oidx[:, 0, :MAXDET]
    scs = osc[:, 0, :MAXDET]
    bxs = jnp.stack([ox1[:, 0, :MAXDET], oy1[:, 0, :MAXDET],
                     ox2[:, 0, :MAXDET], oy2[:, 0, :MAXDET]], axis=-1)
    cls_out = ocls[:, 0, :MAXDET]
    num = onum[:, 0, 0]
    return idxs, scs, bxs, cls_out, num


# speedup vs baseline: 11.5554x; 11.5554x over previous
import functools

import jax
import jax.numpy as jnp
from jax.experimental import pallas as pl
from jax.experimental.pallas import tpu as pltpu

IOU_T = 0.5
SCORE_T = 0.05
MAXDET = 100
NEG = -1e30

LANES = 128
OD = 128


def _nms_tc_body(sc_ref, x1_ref, y1_ref, x2_ref, y2_ref, cls_ref,
                 oidx_ref, osc_ref, ox1_ref, oy1_ref, ox2_ref, oy2_ref,
                 ocls_ref, onum_ref, act_ref):
    rows = sc_ref.shape[1]
    x1 = x1_ref[0]
    y1 = y1_ref[0]
    x2 = x2_ref[0]
    y2 = y2_ref[0]
    cls = cls_ref[0]
    sc = sc_ref[0]
    act_ref[:, :] = jnp.where(sc > SCORE_T, sc, NEG)
    area2 = (x2 - x1) * (y2 - y1)
    r = jax.lax.broadcasted_iota(jnp.int32, (rows, LANES), 0)
    c = jax.lax.broadcasted_iota(jnp.int32, (rows, LANES), 1)
    lin = r * LANES + c
    oiota = jax.lax.broadcasted_iota(jnp.int32, (1, OD), 1)

    def body(t, carry):
        oidx, osc, ox1, oy1, ox2, oy2, ocls, num = carry
        act = act_ref[:, :]
        maxv = jnp.max(act)
        keep = maxv > NEG * 0.5
        idx = jnp.min(jnp.where(act == maxv, lin, jnp.int32(2**30)))
        sel = lin == idx
        zf = jnp.float32(0.0)
        wx1 = jnp.sum(jnp.where(sel, x1, zf))
        wy1 = jnp.sum(jnp.where(sel, y1, zf))
        wx2 = jnp.sum(jnp.where(sel, x2, zf))
        wy2 = jnp.sum(jnp.where(sel, y2, zf))
        wcls = jnp.sum(jnp.where(sel, cls, 0))
        xx1 = jnp.maximum(wx1, x1)
        yy1 = jnp.maximum(wy1, y1)
        xx2 = jnp.minimum(wx2, x2)
        yy2 = jnp.minimum(wy2, y2)
        inter = jnp.maximum(xx2 - xx1, zf) * jnp.maximum(yy2 - yy1, zf)
        area1 = (wx2 - wx1) * (wy2 - wy1)
        iou = inter / (area1 + area2 - inter + jnp.float32(1e-9))
        supp = (iou > IOU_T) | sel
        act_ref[:, :] = jnp.where(supp, NEG, act)
        m = oiota == t
        ki = keep.astype(jnp.int32)
        oidx = jnp.where(m, jnp.where(keep, idx, -1), oidx)
        osc = jnp.where(m, jnp.where(keep, maxv, zf), osc)
        ox1 = jnp.where(m, jnp.where(keep, wx1, zf), ox1)
        oy1 = jnp.where(m, jnp.where(keep, wy1, zf), oy1)
        ox2 = jnp.where(m, jnp.where(keep, wx2, zf), ox2)
        oy2 = jnp.where(m, jnp.where(keep, wy2, zf), oy2)
        ocls = jnp.where(m, jnp.where(keep, wcls, -1), ocls)
        return (oidx, osc, ox1, oy1, ox2, oy2, ocls, num + ki)

    init = (jnp.full((1, OD), -1, jnp.int32),
            jnp.zeros((1, OD), jnp.float32),
            jnp.zeros((1, OD), jnp.float32),
            jnp.zeros((1, OD), jnp.float32),
            jnp.zeros((1, OD), jnp.float32),
            jnp.zeros((1, OD), jnp.float32),
            jnp.full((1, OD), -1, jnp.int32),
            jnp.int32(0))
    oidx, osc, ox1, oy1, ox2, oy2, ocls, num = jax.lax.fori_loop(
        0, MAXDET, body, init)
    oidx_ref[0] = oidx
    osc_ref[0] = osc
    ox1_ref[0] = ox1
    oy1_ref[0] = oy1
    ox2_ref[0] = ox2
    oy2_ref[0] = oy2
    ocls_ref[0] = ocls
    onum_ref[0] = jnp.full((1, OD), num, jnp.int32)


@jax.jit
def kernel(scores, boxes, classes):
    B, N = scores.shape
    rows = (N + LANES - 1) // LANES
    rows = ((rows + 7) // 8) * 8
    NP = rows * LANES
    pad = NP - N
    scp = jnp.pad(scores, ((0, 0), (0, pad)), constant_values=-1.0)
    clsp = jnp.pad(classes, ((0, 0), (0, pad)))
    bx = jnp.pad(boxes, ((0, 0), (0, pad), (0, 0)))
    sc = scp.reshape(B, rows, LANES)
    x1 = bx[:, :, 0].reshape(B, rows, LANES)
    y1 = bx[:, :, 1].reshape(B, rows, LANES)
    x2 = bx[:, :, 2].reshape(B, rows, LANES)
    y2 = bx[:, :, 3].reshape(B, rows, LANES)
    cl = clsp.reshape(B, rows, LANES)

    in_spec = pl.BlockSpec((1, rows, LANES), lambda b: (b, 0, 0))
    out_spec = pl.BlockSpec((1, 1, OD), lambda b: (b, 0, 0))
    of = jax.ShapeDtypeStruct((B, 1, OD), jnp.float32)
    oi = jax.ShapeDtypeStruct((B, 1, OD), jnp.int32)
    outs = pl.pallas_call(
        _nms_tc_body,
        grid=(B,),
        in_specs=[in_spec] * 6,
        out_specs=[out_spec] * 8,
        out_shape=[oi, of, of, of, of, of, oi, oi],
        scratch_shapes=[pltpu.VMEM((rows, LANES), jnp.float32)],
    )(sc, x1, y1, x2, y2, cl)
    oidx, osc, ox1, oy1, ox2, oy2, ocls, onum = outs
    idxs = oidx[:, 0, :MAXDET]
    scs = osc[:, 0, :MAXDET]
    bxs = jnp.stack([ox1[:, 0, :MAXDET], oy1[:, 0, :MAXDET],
                     ox2[:, 0, :MAXDET], oy2[:, 0, :MAXDET]], axis=-1)
    cls_out = ocls[:, 0, :MAXDET]
    num = onum[:, 0, 0]
    return idxs, scs, bxs, cls_out, num
